# Initial kernel scaffold; baseline (speedup 1.0000x reference)
import jax
import jax.numpy as jnp
from jax.experimental import pallas as pl


def kernel(feat, edge_index, distances, interaction_cutoffs, rbf_kernel_means, rbf_kernel_scaling, features_to_use):
    raise NotImplementedError("write your pallas kernel here")


# speedup vs baseline: 6.1854x; 6.1854x over previous
import functools

import jax
import jax.numpy as jnp
import numpy as np
from jax import lax
from jax.experimental import pallas as pl
from jax.experimental.pallas import tpu as pltpu
from jax.experimental.pallas import tpu_sc as plsc

N_NODES = 50000
N_EDGES = 1600000
K = 8
T = 3
LANES = 16
NC = 2
NS = 16
NW = NC * NS
CHUNK = 128
NCH = -(-N_EDGES // (NW * CHUNK))
E_PAD = NW * CHUNK * NCH
NF_HI = -(-N_NODES // (4 * 128))
ROWS = T * N_NODES
ROWS_PAD = 150016
DUMP = 150003


def _he_body(d_ref, ic_ref, rm_ref, rs_ref, o_ref):
    d = d_ref[...]
    for k in range(K):
        ic = ic_ref[k]
        rm = rm_ref[k]
        rs = rs_ref[k]
        rbf = jnp.exp(-rs * (d - rm) ** 2)
        cosv = 0.5 * (jnp.cos(np.pi * d / ic) + 1.0)
        cut = jnp.where(d <= ic, cosv, 0.0)
        o_ref[:, k, :] = rbf * cut


def _compute_he(d2, ic, rm, rs):
    nchunks = d2.shape[0]
    BC = 32
    grid = nchunks // BC
    return pl.pallas_call(
        _he_body,
        grid=(grid,),
        in_specs=[
            pl.BlockSpec((BC, 128), lambda i: (i, 0)),
            pl.BlockSpec(memory_space=pltpu.SMEM),
            pl.BlockSpec(memory_space=pltpu.SMEM),
            pl.BlockSpec(memory_space=pltpu.SMEM),
        ],
        out_specs=pl.BlockSpec((BC, K, 128), lambda i: (i, 0, 0)),
        out_shape=jax.ShapeDtypeStruct((nchunks, K, 128), jnp.float32),
    )(d2, ic, rm, rs)


ZB = 2344


def _sc_body(he_hbm, src_hbm, dst_hbm, feat_hbm, ftu_hbm, z_hbm, out_hbm,
             featb, ftub, hev, rows, srcb, dstb, idxb, xfer, acc):
    cid = lax.axis_index("c")
    sid = lax.axis_index("s")
    wid = sid * NC + cid

    pltpu.sync_copy(feat_hbm, featb)
    pltpu.sync_copy(ftu_hbm, ftub)
    rpt = ROWS_PAD // NS

    def zinit(t, carry):
        r0 = sid * rpt + t * ZB
        pltpu.sync_copy(z_hbm.at[pl.ds(r0, ZB)], xfer)
        pltpu.sync_copy(xfer, acc.at[pl.ds(r0, ZB)])
        return carry

    lax.fori_loop(0, rpt // ZB, zinit, 0)
    plsc.subcore_barrier()

    f0 = ftub[0]
    f1 = ftub[1]
    f2 = ftub[2]
    m255 = jnp.full((16,), 255, jnp.int32)
    iota = lax.iota(jnp.int32, 16)
    dump = jnp.full((16,), DUMP, jnp.int32)

    def chunk_body(i, carry):
        chunk = wid * NCH + i
        pltpu.sync_copy(src_hbm.at[pl.ds(chunk * CHUNK, CHUNK)], srcb)
        pltpu.sync_copy(dst_hbm.at[pl.ds(chunk * CHUNK, CHUNK)], dstb)
        pltpu.sync_copy(he_hbm.at[pl.ds(chunk * K, K)], hev)
        for g in range(CHUNK // 16):
            s = srcb[pl.ds(g * 16, 16)]
            w = plsc.load_gather(featb, [s >> 9, (s >> 2) & 127])
            f = (w >> ((s & 3) << 3)) & m255
            d3 = dstb[pl.ds(g * 16, 16)] * 3
            r = jnp.where(f == f0, d3,
                          jnp.where(f == f1, d3 + 1,
                                    jnp.where(f == f2, d3 + 2, dump)))
            idxb[pl.ds(g * 16, 16)] = r
        for k in range(K):
            ck = jnp.full((16,), k, jnp.int32)
            for g in range(CHUNK // 16):
                v = hev[k, pl.ds(g * 16, 16)]
                plsc.store_scatter(rows, [iota + g * 16, ck], v)
        pltpu.sync_copy(rows, acc.at[idxb], add=True)
        return carry

    lax.fori_loop(0, NCH, chunk_body, 0)

    plsc.subcore_barrier()

    def drain(t, carry):
        r0 = sid * rpt + t * ZB
        pltpu.sync_copy(acc.at[pl.ds(r0, ZB)], xfer)
        pltpu.sync_copy(xfer, out_hbm.at[pl.ds(cid * ROWS_PAD + r0, ZB)])
        return carry

    lax.fori_loop(0, rpt // ZB, drain, 0)


def _sc_scatter(he, src, dst, featv, ftu_b, z):
    mesh = plsc.VectorSubcoreMesh(core_axis_name="c", subcore_axis_name="s",
                                  num_cores=NC, num_subcores=NS)
    f = pl.kernel(
        _sc_body,
        out_type=jax.ShapeDtypeStruct((NC * ROWS_PAD, K), jnp.float32),
        mesh=mesh,
        compiler_params=pltpu.CompilerParams(needs_layout_passes=False,
                                             use_tc_tiling_on_sc=False),
        scratch_types=[
            pltpu.VMEM((NF_HI, 128), jnp.int32),
            pltpu.VMEM((T, 16), jnp.int32),
            pltpu.VMEM((K, CHUNK), jnp.float32),
            pltpu.VMEM((CHUNK, K), jnp.float32),
            pltpu.VMEM((CHUNK,), jnp.int32),
            pltpu.VMEM((CHUNK,), jnp.int32),
            pltpu.VMEM((CHUNK,), jnp.int32),
            pltpu.VMEM((ZB, K), jnp.float32),
            pltpu.VMEM_SHARED((ROWS_PAD, K), jnp.float32),
        ],
    )
    return f(he, src, dst, featv, ftu_b, z)


def _combine_body(a_ref, b_ref, o_ref):
    o_ref[...] = a_ref[...] + b_ref[...]


def _combine(p0, p1):
    n = p0.shape[0]
    BC = 32
    return pl.pallas_call(
        _combine_body,
        grid=(n // BC,),
        in_specs=[pl.BlockSpec((BC, 128), lambda i: (i, 0)),
                  pl.BlockSpec((BC, 128), lambda i: (i, 0))],
        out_specs=pl.BlockSpec((BC, 128), lambda i: (i, 0)),
        out_shape=jax.ShapeDtypeStruct((n, 128), jnp.float32),
    )(p0, p1)


def kernel(feat, edge_index, distances, interaction_cutoffs, rbf_kernel_means,
           rbf_kernel_scaling, features_to_use):
    ei = edge_index.astype(jnp.int32)
    pad = E_PAD - N_EDGES
    src = jnp.pad(ei[0], (0, pad))
    dst = jnp.pad(ei[1], (0, pad), constant_values=N_NODES)
    d2 = jnp.pad(distances[:, 0], (0, pad), constant_values=0.5)
    d2 = d2.reshape(E_PAD // 128, 128)

    he = _compute_he(d2, interaction_cutoffs, rbf_kernel_means,
                     rbf_kernel_scaling)
    he = he.reshape(-1, 128)

    fi = jnp.pad(feat[:, 0].astype(jnp.int32), (0, NF_HI * 512 - N_NODES),
                 constant_values=-1).reshape(-1, 4)
    featv = ((fi[:, 0] & 255) | ((fi[:, 1] & 255) << 8)
             | ((fi[:, 2] & 255) << 16) | ((fi[:, 3] & 255) << 24))
    featv = featv.reshape(NF_HI, 128)
    ftu_b = jnp.broadcast_to(features_to_use.astype(jnp.int32)[:, None],
                             (T, 16))
    z = jnp.zeros((ROWS_PAD, K), jnp.float32)
    partial = _sc_scatter(he, src, dst, featv, ftu_b, z)

    partial = partial.reshape(NC, ROWS_PAD * K // 128, 128)
    p0 = partial[0]
    p1 = partial[1]
    out = _combine(p0, p1)
    return out.reshape(-1)[: ROWS * K].reshape(N_NODES, T * K)


# speedup vs baseline: 6.5365x; 1.0568x over previous
import jax
import jax.numpy as jnp
import numpy as np
from jax import lax
from jax.experimental import pallas as pl
from jax.experimental.pallas import tpu as pltpu
from jax.experimental.pallas import tpu_sc as plsc

N_NODES = 50000
N_EDGES = 1600000
K = 8
T = 3
NC = 2
NS = 16
NW = NC * NS
CHUNK = 128
NCH = 392
E_PAD = NW * CHUNK * NCH
NCHT = E_PAD // CHUNK
NF_HI = -(-N_NODES // (4 * 128))
ROWS = T * N_NODES
ROWS_PAD = 151552
DUMP = 150003
RPT = ROWS_PAD // NS
ZB = 592


def _he_body(d_ref, ic_ref, rm_ref, rs_ref, o_ref):
    d = d_ref[...]
    for k in range(K):
        ic = ic_ref[k]
        rm = rm_ref[k]
        rs = rs_ref[k]
        rbf = jnp.exp(-rs * (d - rm) ** 2)
        cosv = 0.5 * (jnp.cos(np.pi * d / ic) + 1.0)
        cut = jnp.where(d <= ic, cosv, 0.0)
        o_ref[:, k, :] = rbf * cut


def _compute_he(d2, ic, rm, rs):
    BC = 32
    return pl.pallas_call(
        _he_body,
        grid=(NCHT // BC,),
        in_specs=[
            pl.BlockSpec((BC, 128), lambda i: (i, 0)),
            pl.BlockSpec(memory_space=pltpu.SMEM),
            pl.BlockSpec(memory_space=pltpu.SMEM),
            pl.BlockSpec(memory_space=pltpu.SMEM),
        ],
        out_specs=pl.BlockSpec((BC, K, 128), lambda i: (i, 0, 0)),
        out_shape=jax.ShapeDtypeStruct((NCHT, K, 128), jnp.float32),
    )(d2, ic, rm, rs)


def _sc_body(he_hbm, sd_hbm, feat_hbm, ftu_hbm, z_hbm, out_hbm,
             featb, ftub, hev0, hev1, sdb0, sdb1, rows0, rows1, idx0, idx1,
             xfer, acc, dsem0, dsem1):
    cid = lax.axis_index("c")
    sid = lax.axis_index("s")
    wid = sid * NC + cid
    base = wid * NCH

    hev = (hev0, hev1)
    sdb = (sdb0, sdb1)
    rows = (rows0, rows1)
    idx = (idx0, idx1)
    dsem = (dsem0, dsem1)

    pltpu.sync_copy(feat_hbm, featb)
    pltpu.sync_copy(ftu_hbm, ftub)

    def zinit(t, carry):
        r0 = sid * RPT + t * ZB
        pltpu.sync_copy(z_hbm.at[pl.ds(r0, ZB)], xfer)
        pltpu.sync_copy(xfer, acc.at[pl.ds(r0, ZB)])
        return carry

    lax.fori_loop(0, RPT // ZB, zinit, 0)
    plsc.subcore_barrier()

    f0 = ftub[pl.ds(0, 16)]
    f1 = ftub[pl.ds(16, 16)]
    f2 = ftub[pl.ds(32, 16)]
    iota = lax.iota(jnp.int32, 16)
    dump = jnp.full((16,), DUMP, jnp.int32)
    m255 = jnp.full((16,), 255, jnp.int32)

    def start_in(c, p):
        pltpu.async_copy(he_hbm.at[pl.ds(c * K, K)], hev[p], dsem[p])
        pltpu.async_copy(sd_hbm.at[pl.ds(c * 2 * CHUNK, 2 * CHUNK)], sdb[p],
                         dsem[p])

    start_in(base + 0, 0)
    start_in(base + 1, 1)

    def pair_body(i, carry):
        for p in range(2):
            cl = 2 * i + p
            c = base + cl
            pltpu.make_async_copy(he_hbm.at[pl.ds(c * K, K)], hev[p],
                                  dsem[p]).wait()
            pltpu.make_async_copy(sd_hbm.at[pl.ds(c * 2 * CHUNK, 2 * CHUNK)],
                                  sdb[p], dsem[p]).wait()

            for g in range(CHUNK // 16):
                s = plsc.bitcast(sdb[p][pl.ds(g * 16, 16)], jnp.int32)
                d = plsc.bitcast(sdb[p][pl.ds(CHUNK + g * 16, 16)], jnp.int32)
                w = plsc.load_gather(featb, [s >> 9, (s >> 2) & 127])
                f = (w >> ((s & 3) << 3)) & m255
                d3 = d * 3
                r = jnp.where(f == f0, d3,
                              jnp.where(f == f1, d3 + 1,
                                        jnp.where(f == f2, d3 + 2, dump)))
                idx[p][pl.ds(g * 16, 16)] = r

            for k in range(K):
                ck = jnp.full((16,), k, jnp.int32)
                for g in range(CHUNK // 16):
                    v = hev[p][k, pl.ds(g * 16, 16)]
                    plsc.store_scatter(rows[p], [iota + g * 16, ck], v)

            @pl.when(cl + 2 < NCH)
            def _():
                start_in(c + 2, p)

            pltpu.sync_copy(rows[p], acc.at[idx[p]], add=True)
        return carry

    lax.fori_loop(0, NCH // 2, pair_body, 0)

    plsc.subcore_barrier()

    def drain(t, carry):
        r0 = sid * RPT + t * ZB
        pltpu.sync_copy(acc.at[pl.ds(r0, ZB)], xfer)
        pltpu.sync_copy(xfer, out_hbm.at[pl.ds(cid * ROWS_PAD + r0, ZB)])
        return carry

    lax.fori_loop(0, RPT // ZB, drain, 0)


def _sc_scatter(he, sd, featv, ftu_b, z):
    mesh = plsc.VectorSubcoreMesh(core_axis_name="c", subcore_axis_name="s",
                                  num_cores=NC, num_subcores=NS)
    f = pl.kernel(
        _sc_body,
        out_type=jax.ShapeDtypeStruct((NC * ROWS_PAD, K), jnp.float32),
        mesh=mesh,
        compiler_params=pltpu.CompilerParams(needs_layout_passes=False,
                                             use_tc_tiling_on_sc=False),
        scratch_types=[
            pltpu.VMEM((NF_HI, 128), jnp.int32),
            pltpu.VMEM((3 * 16,), jnp.int32),
            pltpu.VMEM((K, CHUNK), jnp.float32),
            pltpu.VMEM((K, CHUNK), jnp.float32),
            pltpu.VMEM((2 * CHUNK,), jnp.float32),
            pltpu.VMEM((2 * CHUNK,), jnp.float32),
            pltpu.VMEM((CHUNK, K), jnp.float32),
            pltpu.VMEM((CHUNK, K), jnp.float32),
            pltpu.VMEM((CHUNK,), jnp.int32),
            pltpu.VMEM((CHUNK,), jnp.int32),
            pltpu.VMEM((ZB, K), jnp.float32),
            pltpu.VMEM_SHARED((ROWS_PAD, K), jnp.float32),
            pltpu.SemaphoreType.DMA,
            pltpu.SemaphoreType.DMA,
        ],
    )
    return f(he, sd, featv, ftu_b, z)


def _combine_body(a_ref, b_ref, o_ref):
    o_ref[...] = a_ref[...] + b_ref[...]


def _combine(p0, p1):
    n = p0.shape[0]
    BC = 32
    return pl.pallas_call(
        _combine_body,
        grid=(n // BC,),
        in_specs=[pl.BlockSpec((BC, 128), lambda i: (i, 0)),
                  pl.BlockSpec((BC, 128), lambda i: (i, 0))],
        out_specs=pl.BlockSpec((BC, 128), lambda i: (i, 0)),
        out_shape=jax.ShapeDtypeStruct((n, 128), jnp.float32),
    )(p0, p1)


def kernel(feat, edge_index, distances, interaction_cutoffs, rbf_kernel_means,
           rbf_kernel_scaling, features_to_use):
    ei = edge_index.astype(jnp.int32)
    pad = E_PAD - N_EDGES
    src = jnp.pad(ei[0], (0, pad))
    dst = jnp.pad(ei[1], (0, pad), constant_values=N_NODES)
    sd = jnp.stack([src.reshape(NCHT, CHUNK), dst.reshape(NCHT, CHUNK)],
                   axis=1).reshape(-1)
    sd = lax.bitcast_convert_type(sd, jnp.float32)
    d2 = jnp.pad(distances[:, 0], (0, pad), constant_values=0.5)
    d2 = d2.reshape(NCHT, 128)

    he = _compute_he(d2, interaction_cutoffs, rbf_kernel_means,
                     rbf_kernel_scaling)
    he = he.reshape(NCHT * K, 128)

    fi = jnp.pad(feat[:, 0].astype(jnp.int32), (0, NF_HI * 512 - N_NODES),
                 constant_values=-1).reshape(-1, 4)
    featv = ((fi[:, 0] & 255) | ((fi[:, 1] & 255) << 8)
             | ((fi[:, 2] & 255) << 16) | ((fi[:, 3] & 255) << 24))
    featv = featv.reshape(NF_HI, 128)
    ftu_b = jnp.broadcast_to(features_to_use.astype(jnp.int32)[:, None],
                             (T, 16)).reshape(-1)
    z = jnp.zeros((ROWS_PAD, K), jnp.float32)

    partial = _sc_scatter(he, sd, featv, ftu_b, z)

    partial = partial.reshape(NC, ROWS_PAD * K // 128, 128)
    out = _combine(partial[0], partial[1])
    return out.reshape(-1)[: ROWS * K].reshape(N_NODES, T * K)
